# baseline (speedup 1.0000x reference)
import functools

import jax
import jax.numpy as jnp
from jax import lax
from jax.experimental import pallas as pl
from jax.experimental.pallas import tpu as pltpu
from jax.experimental.pallas import tpu_sc as plsc

N = 10000
E = 320000
F = 128
NB = 3
G = 64

NPAD = 10240
ROWS_PER_SUB = NPAD // 16
NW = 32
EPW = E // NW
CH = 40
NCHUNK = EPW // CH
CHD = 80
NCHUNKD = EPW // CHD

_EA_CHUNK = 512
_EA_GRID = E // _EA_CHUNK
_NROW = 1000
_NGRID = N // _NROW


def _edge_feat_body(ea_ref, cen_ref, wcat_ref, bcat_ref, o0_ref, o1_ref, o2_ref):
    d = ea_ref[0]
    t = (cen_ref[...] - d) * (99.0 / 5.0)
    e = jnp.exp(-(t * t))
    outs = (o0_ref, o1_ref, o2_ref)
    for b in range(NB):
        outs[b][...] = lax.dot_general(
            e, wcat_ref[b], (((0,), (0,)), ((), ())),
            preferred_element_type=jnp.float32) + bcat_ref[b]


def _edge_feats(ea3, centers, wcat, bcat):
    out = jax.ShapeDtypeStruct((E, 2 * F), jnp.float32)
    return pl.pallas_call(
        _edge_feat_body,
        grid=(_EA_GRID,),
        in_specs=[
            pl.BlockSpec((1, 1, _EA_CHUNK), lambda i: (i, 0, 0)),
            pl.BlockSpec((128, 1), lambda i: (0, 0)),
            pl.BlockSpec((NB, 128, 2 * F), lambda i: (0, 0, 0)),
            pl.BlockSpec((NB, 1, 2 * F), lambda i: (0, 0, 0)),
        ],
        out_specs=[pl.BlockSpec((_EA_CHUNK, 2 * F), lambda i: (i, 0))] * NB,
        out_shape=[out, out, out],
    )(ea3, centers, wcat, bcat)


def _tables0_body(x_ref, wd_ref, wsrc_ref, dt_ref, st_ref):
    xb = x_ref[...]
    dt_ref[...] = jnp.dot(xb, wd_ref[...], preferred_element_type=jnp.float32)
    st_ref[...] = jnp.dot(xb, wsrc_ref[...], preferred_element_type=jnp.float32)


def _tables0(x, wd, wsrc):
    out = jax.ShapeDtypeStruct((N, 2 * F), jnp.float32)
    return pl.pallas_call(
        _tables0_body,
        grid=(_NGRID,),
        in_specs=[
            pl.BlockSpec((_NROW, F), lambda i: (i, 0)),
            pl.BlockSpec((F, 2 * F), lambda i: (0, 0)),
            pl.BlockSpec((F, 2 * F), lambda i: (0, 0)),
        ],
        out_specs=[pl.BlockSpec((_NROW, 2 * F), lambda i: (i, 0))] * 2,
        out_shape=[out, out],
    )(x, wd, wsrc)


def _tables_bn_body(x_ref, agg_ref, ssum_ref, ssq_ref, gam_ref, bet_ref,
                    wd_ref, wsrc_ref, xn_ref, dt_ref, st_ref):
    mu = ssum_ref[0:1, :] * (1.0 / N)
    ex2 = ssq_ref[0:1, :] * (1.0 / N)
    scv = gam_ref[...] * lax.rsqrt(ex2 - mu * mu + 1e-5)
    xn = x_ref[...] + (agg_ref[...] - mu) * scv + bet_ref[...]
    xn_ref[...] = xn
    dt_ref[...] = jnp.dot(xn, wd_ref[...], preferred_element_type=jnp.float32)
    st_ref[...] = jnp.dot(xn, wsrc_ref[...], preferred_element_type=jnp.float32)


def _tables_bn(x, agg, ssum, ssq, gam, bet, wd, wsrc):
    out2 = jax.ShapeDtypeStruct((N, 2 * F), jnp.float32)
    outx = jax.ShapeDtypeStruct((N, F), jnp.float32)
    return pl.pallas_call(
        _tables_bn_body,
        grid=(_NGRID,),
        in_specs=[
            pl.BlockSpec((_NROW, F), lambda i: (i, 0)),
            pl.BlockSpec((_NROW, F), lambda i: (i, 0)),
            pl.BlockSpec((8, F), lambda i: (0, 0)),
            pl.BlockSpec((8, F), lambda i: (0, 0)),
            pl.BlockSpec((1, F), lambda i: (0, 0)),
            pl.BlockSpec((1, F), lambda i: (0, 0)),
            pl.BlockSpec((F, 2 * F), lambda i: (0, 0)),
            pl.BlockSpec((F, 2 * F), lambda i: (0, 0)),
        ],
        out_specs=[
            pl.BlockSpec((_NROW, F), lambda i: (i, 0)),
            pl.BlockSpec((_NROW, 2 * F), lambda i: (i, 0)),
            pl.BlockSpec((_NROW, 2 * F), lambda i: (i, 0)),
        ],
        out_shape=[outx, out2, out2],
    )(x, agg, ssum, ssq, gam, bet, wd, wsrc)


def _agg_stats_body(p_ref, dp_ref, agg_ref, ssum_ref, ssq_ref):
    i = pl.program_id(0)
    deg = jnp.maximum(dp_ref[0, :, 0:1] + dp_ref[1, :, 0:1], 1.0)
    agg = (p_ref[0] + p_ref[1]) / deg
    agg_ref[...] = agg
    s = jnp.sum(agg, axis=0, keepdims=True)
    s2 = jnp.sum(agg * agg, axis=0, keepdims=True)

    @pl.when(i == 0)
    def _():
        ssum_ref[...] = jnp.zeros_like(ssum_ref)
        ssq_ref[...] = jnp.zeros_like(ssq_ref)

    ssum_ref[...] += jnp.broadcast_to(s, (8, F))
    ssq_ref[...] += jnp.broadcast_to(s2, (8, F))


def _agg_stats(aggp, degp):
    return pl.pallas_call(
        _agg_stats_body,
        grid=(_NGRID,),
        in_specs=[
            pl.BlockSpec((2, _NROW, F), lambda i: (0, i, 0)),
            pl.BlockSpec((2, _NROW, F), lambda i: (0, i, 0)),
        ],
        out_specs=[
            pl.BlockSpec((_NROW, F), lambda i: (i, 0)),
            pl.BlockSpec((8, F), lambda i: (0, 0)),
            pl.BlockSpec((8, F), lambda i: (0, 0)),
        ],
        out_shape=[
            jax.ShapeDtypeStruct((N, F), jnp.float32),
            jax.ShapeDtypeStruct((8, F), jnp.float32),
            jax.ShapeDtypeStruct((8, F), jnp.float32),
        ],
    )(aggp, degp)


def _final_body(x_ref, agg_ref, ssum_ref, ssq_ref, gam_ref, bet_ref,
                boh_ref, bt_ref, w1_ref, b1_ref, w2_ref, b2_ref,
                w3_ref, b3_ref, out_ref, sum_s, max_s, cnt_s):
    i = pl.program_id(0)
    mu = ssum_ref[0:1, :] * (1.0 / N)
    ex2 = ssq_ref[0:1, :] * (1.0 / N)
    scv = gam_ref[...] * lax.rsqrt(ex2 - mu * mu + 1e-5)
    xf = x_ref[...] + (agg_ref[...] - mu) * scv + bet_ref[...]

    br = boh_ref[0]
    bc = bt_ref[0]
    g_iota = lax.broadcasted_iota(jnp.int32, (G, 1), 0)
    oht = (g_iota == br).astype(jnp.float32)

    @pl.when(i == 0)
    def _():
        sum_s[...] = jnp.zeros_like(sum_s)
        cnt_s[...] = jnp.zeros_like(cnt_s)
        max_s[...] = jnp.full_like(max_s, -3.4e38)

    sum_s[...] += jnp.dot(oht, xf, preferred_element_type=jnp.float32)
    cnt_s[...] += jnp.sum(oht, axis=1, keepdims=True)
    for g in range(G):
        xm = jnp.where(bc == g, xf, -3.4e38)
        max_s[g:g + 1, :] = jnp.maximum(
            max_s[g:g + 1, :], jnp.max(xm, axis=0, keepdims=True))

    @pl.when(i == _NGRID - 1)
    def _():
        cnt = jnp.maximum(cnt_s[...], 1.0)
        mean_p = sum_s[...] / cnt
        h = jnp.concatenate([mean_p, max_s[...], sum_s[...]], axis=1)
        h = jnp.maximum(
            jnp.dot(h, w1_ref[...], preferred_element_type=jnp.float32)
            + b1_ref[...], 0.0)
        h = jnp.maximum(
            jnp.dot(h, w2_ref[...], preferred_element_type=jnp.float32)
            + b2_ref[...], 0.0)
        out_ref[...] = (jnp.dot(h, w3_ref[...], preferred_element_type=jnp.float32)
                        + b3_ref[...])


def _final(x, agg, ssum, ssq, gam, bet, boh, bt, w1, b1, w2, b2, w3, b3):
    return pl.pallas_call(
        _final_body,
        grid=(_NGRID,),
        in_specs=[
            pl.BlockSpec((_NROW, F), lambda i: (i, 0)),
            pl.BlockSpec((_NROW, F), lambda i: (i, 0)),
            pl.BlockSpec((8, F), lambda i: (0, 0)),
            pl.BlockSpec((8, F), lambda i: (0, 0)),
            pl.BlockSpec((1, F), lambda i: (0, 0)),
            pl.BlockSpec((1, F), lambda i: (0, 0)),
            pl.BlockSpec((1, 1, _NROW), lambda i: (i, 0, 0)),
            pl.BlockSpec((1, _NROW, 1), lambda i: (i, 0, 0)),
            pl.BlockSpec((3 * F, 32), lambda i: (0, 0)),
            pl.BlockSpec((1, 32), lambda i: (0, 0)),
            pl.BlockSpec((32, 16), lambda i: (0, 0)),
            pl.BlockSpec((1, 16), lambda i: (0, 0)),
            pl.BlockSpec((16, 1), lambda i: (0, 0)),
            pl.BlockSpec((1, 1), lambda i: (0, 0)),
        ],
        out_specs=pl.BlockSpec((G, 1), lambda i: (0, 0)),
        out_shape=jax.ShapeDtypeStruct((G, 1), jnp.float32),
        scratch_shapes=[
            pltpu.VMEM((G, F), jnp.float32),
            pltpu.VMEM((G, F), jnp.float32),
            pltpu.VMEM((G, 1), jnp.float32),
        ],
    )(x, agg, ssum, ssq, gam, bet, boh, bt, w1, b1, w2, b2, w3, b3)


def _gate_row(rows_d, rows_s, ef_v, msg_v, r):
    for v in range(F // 16):
        o = v * 16
        fa = rows_d[r, pl.ds(o, 16)] + rows_s[r, pl.ds(o, 16)] \
            + ef_v[r, pl.ds(o, 16)]
        sb = rows_d[r, pl.ds(F + o, 16)] + rows_s[r, pl.ds(F + o, 16)] \
            + ef_v[r, pl.ds(F + o, 16)]
        sg = 1.0 / (1.0 + jnp.exp(-fa))
        t = jnp.exp(-jnp.abs(sb))
        u = t / (2.0 + t)
        u2 = u * u
        ln1p = 2.0 * u * (1.0 + u2 * ((1.0 / 3.0) + u2 * ((1.0 / 5.0)
                          + u2 * ((1.0 / 7.0) + u2 * (1.0 / 9.0)))))
        sp = jnp.maximum(sb, 0.0) + ln1p
        msg_v[r, pl.ds(o, 16)] = sg * sp


@functools.lru_cache(maxsize=None)
def _make_sc_agg_kernel():
    mesh = plsc.VectorSubcoreMesh(core_axis_name="c", subcore_axis_name="s")
    scratch = [
        pltpu.VMEM((CH,), jnp.int32),
        pltpu.VMEM((CH,), jnp.int32),
        pltpu.VMEM((CH, 2 * F), jnp.float32),
        pltpu.VMEM((CH, 2 * F), jnp.float32),
        pltpu.VMEM((CH, 2 * F), jnp.float32),
        pltpu.VMEM((CH, F), jnp.float32),
        pltpu.VMEM_SHARED((NPAD, F), jnp.float32),
        pltpu.SemaphoreType.DMA,
        pltpu.SemaphoreType.DMA,
        pltpu.SemaphoreType.DMA,
    ]

    def body(dt_hbm, st_hbm, ef_hbm, di_hbm, si_hbm, z128_hbm, aggp_hbm,
             di_v, si_v, rows_d, rows_s, ef_v, msg_v, agg_sh,
             sem1, sem2, sem3):
        c = lax.axis_index("c")
        s = lax.axis_index("s")
        wid = s * 2 + c
        pltpu.sync_copy(z128_hbm, agg_sh.at[pl.ds(s * ROWS_PER_SUB, ROWS_PER_SUB)])
        plsc.subcore_barrier()

        def chunk(i, carry):
            base = wid * EPW + i * CH
            pltpu.sync_copy(di_hbm.at[pl.ds(base, CH)], di_v)
            pltpu.sync_copy(si_hbm.at[pl.ds(base, CH)], si_v)
            cp1 = pltpu.async_copy(dt_hbm.at[di_v], rows_d, sem1)
            cp2 = pltpu.async_copy(st_hbm.at[si_v], rows_s, sem2)
            cp3 = pltpu.async_copy(ef_hbm.at[pl.ds(base, CH)], ef_v, sem3)
            cp1.wait()
            cp2.wait()
            cp3.wait()

            def row(r, cc):
                _gate_row(rows_d, rows_s, ef_v, msg_v, r)
                return cc

            lax.fori_loop(0, CH, row, 0)
            pltpu.sync_copy(msg_v, agg_sh.at[di_v], add=True)
            return carry

        lax.fori_loop(0, NCHUNK, chunk, 0)
        plsc.subcore_barrier()
        sl = pl.ds(s * ROWS_PER_SUB, ROWS_PER_SUB)
        pltpu.sync_copy(agg_sh.at[sl], aggp_hbm.at[c, sl])

    return pl.kernel(body, out_type=jax.ShapeDtypeStruct((2, NPAD, F), jnp.float32),
                     mesh=mesh, scratch_types=tuple(scratch))


@functools.lru_cache(maxsize=None)
def _make_sc_deg_kernel():
    mesh = plsc.VectorSubcoreMesh(core_axis_name="c", subcore_axis_name="s")
    scratch = [
        pltpu.VMEM((CHD,), jnp.int32),
        pltpu.VMEM((CHD, F), jnp.float32),
        pltpu.VMEM_SHARED((NPAD, F), jnp.float32),
    ]

    def body(di_hbm, z128_hbm, ones_hbm, degp_hbm, di_v, ones_v, deg_sh):
        c = lax.axis_index("c")
        s = lax.axis_index("s")
        wid = s * 2 + c
        pltpu.sync_copy(z128_hbm, deg_sh.at[pl.ds(s * ROWS_PER_SUB, ROWS_PER_SUB)])
        pltpu.sync_copy(ones_hbm, ones_v)
        plsc.subcore_barrier()

        def chunk(i, carry):
            base = wid * EPW + i * CHD
            pltpu.sync_copy(di_hbm.at[pl.ds(base, CHD)], di_v)
            pltpu.sync_copy(ones_v, deg_sh.at[di_v], add=True)
            return carry

        lax.fori_loop(0, NCHUNKD, chunk, 0)
        plsc.subcore_barrier()
        sl = pl.ds(s * ROWS_PER_SUB, ROWS_PER_SUB)
        pltpu.sync_copy(deg_sh.at[sl], degp_hbm.at[c, sl])

    return pl.kernel(body, out_type=jax.ShapeDtypeStruct((2, NPAD, F), jnp.float32),
                     mesh=mesh, scratch_types=tuple(scratch))


def _sc_deg(dsti, z128, ones):
    return _make_sc_deg_kernel()(dsti, z128, ones)


def _sc_agg(dt, st, ef, dsti, srci, z128):
    return _make_sc_agg_kernel()(dt, st, ef, dsti, srci, z128)


def kernel(x, edge_index, edge_attr, batch, Wf, bf, Ws, bs,
           bn_gamma, bn_beta, W1, b1, W2, b2, W3, b3):
    f32 = jnp.float32
    src = edge_index[0]
    dst = edge_index[1]

    centers = jnp.concatenate(
        [jnp.linspace(0.0, 5.0, 100, dtype=f32),
         jnp.full((28,), 1e9, dtype=f32)]).reshape(128, 1)
    wcat = jnp.pad(jnp.concatenate([Wf[:, 2 * F:, :], Ws[:, 2 * F:, :]], axis=2),
                   ((0, 0), (0, 28), (0, 0)))
    bcat = jnp.concatenate([bf, bs], axis=1).reshape(NB, 1, 2 * F)
    wd = jnp.concatenate([Wf[:, :F, :], Ws[:, :F, :]], axis=2)
    wsrc = jnp.concatenate([Wf[:, F:2 * F, :], Ws[:, F:2 * F, :]], axis=2)

    ea3 = edge_attr.reshape(_EA_GRID, 1, _EA_CHUNK)
    efs = _edge_feats(ea3, centers, wcat, bcat)

    z128 = jnp.zeros((ROWS_PER_SUB, F), f32)
    ones128 = jnp.ones((CHD, F), f32)

    boh = batch.reshape(_NGRID, 1, _NROW)
    bt = batch.reshape(_NGRID, _NROW, 1)

    degp = _sc_deg(dst, z128, ones128)
    xcur = x
    aggp = None
    for b in range(NB):
        if b == 0:
            dt, st = _tables0(xcur, wd[0], wsrc[0])
        else:
            agg, ssum, ssq = _agg_stats(aggp, degp)
            xcur, dt, st = _tables_bn(
                xcur, agg, ssum, ssq,
                bn_gamma[b - 1].reshape(1, F), bn_beta[b - 1].reshape(1, F),
                wd[b], wsrc[b])
        aggp = _sc_agg(dt, st, efs[b], dst, src, z128)

    agg, ssum, ssq = _agg_stats(aggp, degp)
    out = _final(xcur, agg, ssum, ssq,
                 bn_gamma[2].reshape(1, F), bn_beta[2].reshape(1, F),
                 boh, bt, W1, b1.reshape(1, 32), W2, b2.reshape(1, 16),
                 W3, b3.reshape(1, 1))
    return out


# speedup vs baseline: 1.2728x; 1.2728x over previous
import functools

import jax
import jax.numpy as jnp
from jax import lax
from jax.experimental import pallas as pl
from jax.experimental.pallas import tpu as pltpu
from jax.experimental.pallas import tpu_sc as plsc

N = 10000
E = 320000
F = 128
NB = 3
G = 64

NPAD = 10240
ROWS_PER_SUB = NPAD // 16
NW = 32
EPW = E // NW
CH = 16
NCHUNK = EPW // CH
CHD = 80
NCHUNKD = EPW // CHD

_EA_CHUNK = 512
_EA_GRID = E // _EA_CHUNK
_NROW = 1000
_NGRID = N // _NROW


def _edge_feat_body(ea_ref, cen_ref, wcat_ref, bcat_ref, o0_ref, o1_ref, o2_ref):
    d = ea_ref[0]
    t = (cen_ref[...] - d) * (99.0 / 5.0)
    e = jnp.exp(-(t * t))
    outs = (o0_ref, o1_ref, o2_ref)
    for b in range(NB):
        outs[b][...] = lax.dot_general(
            e, wcat_ref[b], (((0,), (0,)), ((), ())),
            preferred_element_type=jnp.float32) + bcat_ref[b]


def _edge_feats(ea3, centers, wcat, bcat):
    out = jax.ShapeDtypeStruct((E, 2 * F), jnp.float32)
    return pl.pallas_call(
        _edge_feat_body,
        grid=(_EA_GRID,),
        in_specs=[
            pl.BlockSpec((1, 1, _EA_CHUNK), lambda i: (i, 0, 0)),
            pl.BlockSpec((128, 1), lambda i: (0, 0)),
            pl.BlockSpec((NB, 128, 2 * F), lambda i: (0, 0, 0)),
            pl.BlockSpec((NB, 1, 2 * F), lambda i: (0, 0, 0)),
        ],
        out_specs=[pl.BlockSpec((_EA_CHUNK, 2 * F), lambda i: (i, 0))] * NB,
        out_shape=[out, out, out],
    )(ea3, centers, wcat, bcat)


def _tables0_body(x_ref, wd_ref, wsrc_ref, dt_ref, st_ref):
    xb = x_ref[...]
    dt_ref[...] = jnp.dot(xb, wd_ref[...], preferred_element_type=jnp.float32)
    st_ref[...] = jnp.dot(xb, wsrc_ref[...], preferred_element_type=jnp.float32)


def _tables0(x, wd, wsrc):
    out = jax.ShapeDtypeStruct((N, 2 * F), jnp.float32)
    return pl.pallas_call(
        _tables0_body,
        grid=(_NGRID,),
        in_specs=[
            pl.BlockSpec((_NROW, F), lambda i: (i, 0)),
            pl.BlockSpec((F, 2 * F), lambda i: (0, 0)),
            pl.BlockSpec((F, 2 * F), lambda i: (0, 0)),
        ],
        out_specs=[pl.BlockSpec((_NROW, 2 * F), lambda i: (i, 0))] * 2,
        out_shape=[out, out],
    )(x, wd, wsrc)


def _tables_bn_body(x_ref, agg_ref, ssum_ref, ssq_ref, gam_ref, bet_ref,
                    wd_ref, wsrc_ref, xn_ref, dt_ref, st_ref):
    mu = ssum_ref[0:1, :] * (1.0 / N)
    ex2 = ssq_ref[0:1, :] * (1.0 / N)
    scv = gam_ref[...] * lax.rsqrt(ex2 - mu * mu + 1e-5)
    xn = x_ref[...] + (agg_ref[...] - mu) * scv + bet_ref[...]
    xn_ref[...] = xn
    dt_ref[...] = jnp.dot(xn, wd_ref[...], preferred_element_type=jnp.float32)
    st_ref[...] = jnp.dot(xn, wsrc_ref[...], preferred_element_type=jnp.float32)


def _tables_bn(x, agg, ssum, ssq, gam, bet, wd, wsrc):
    out2 = jax.ShapeDtypeStruct((N, 2 * F), jnp.float32)
    outx = jax.ShapeDtypeStruct((N, F), jnp.float32)
    return pl.pallas_call(
        _tables_bn_body,
        grid=(_NGRID,),
        in_specs=[
            pl.BlockSpec((_NROW, F), lambda i: (i, 0)),
            pl.BlockSpec((_NROW, F), lambda i: (i, 0)),
            pl.BlockSpec((8, F), lambda i: (0, 0)),
            pl.BlockSpec((8, F), lambda i: (0, 0)),
            pl.BlockSpec((1, F), lambda i: (0, 0)),
            pl.BlockSpec((1, F), lambda i: (0, 0)),
            pl.BlockSpec((F, 2 * F), lambda i: (0, 0)),
            pl.BlockSpec((F, 2 * F), lambda i: (0, 0)),
        ],
        out_specs=[
            pl.BlockSpec((_NROW, F), lambda i: (i, 0)),
            pl.BlockSpec((_NROW, 2 * F), lambda i: (i, 0)),
            pl.BlockSpec((_NROW, 2 * F), lambda i: (i, 0)),
        ],
        out_shape=[outx, out2, out2],
    )(x, agg, ssum, ssq, gam, bet, wd, wsrc)


def _agg_stats_body(p_ref, dp_ref, agg_ref, ssum_ref, ssq_ref):
    i = pl.program_id(0)
    deg = jnp.maximum(dp_ref[0, :, 0:1] + dp_ref[1, :, 0:1], 1.0)
    agg = (p_ref[0] + p_ref[1]) / deg
    agg_ref[...] = agg
    s = jnp.sum(agg, axis=0, keepdims=True)
    s2 = jnp.sum(agg * agg, axis=0, keepdims=True)

    @pl.when(i == 0)
    def _():
        ssum_ref[...] = jnp.zeros_like(ssum_ref)
        ssq_ref[...] = jnp.zeros_like(ssq_ref)

    ssum_ref[...] += jnp.broadcast_to(s, (8, F))
    ssq_ref[...] += jnp.broadcast_to(s2, (8, F))


def _agg_stats(aggp, degp):
    return pl.pallas_call(
        _agg_stats_body,
        grid=(_NGRID,),
        in_specs=[
            pl.BlockSpec((2, _NROW, F), lambda i: (0, i, 0)),
            pl.BlockSpec((2, _NROW, F), lambda i: (0, i, 0)),
        ],
        out_specs=[
            pl.BlockSpec((_NROW, F), lambda i: (i, 0)),
            pl.BlockSpec((8, F), lambda i: (0, 0)),
            pl.BlockSpec((8, F), lambda i: (0, 0)),
        ],
        out_shape=[
            jax.ShapeDtypeStruct((N, F), jnp.float32),
            jax.ShapeDtypeStruct((8, F), jnp.float32),
            jax.ShapeDtypeStruct((8, F), jnp.float32),
        ],
    )(aggp, degp)


def _final_body(x_ref, agg_ref, ssum_ref, ssq_ref, gam_ref, bet_ref,
                boh_ref, bt_ref, w1_ref, b1_ref, w2_ref, b2_ref,
                w3_ref, b3_ref, out_ref, sum_s, max_s, cnt_s):
    i = pl.program_id(0)
    mu = ssum_ref[0:1, :] * (1.0 / N)
    ex2 = ssq_ref[0:1, :] * (1.0 / N)
    scv = gam_ref[...] * lax.rsqrt(ex2 - mu * mu + 1e-5)
    xf = x_ref[...] + (agg_ref[...] - mu) * scv + bet_ref[...]

    br = boh_ref[0]
    bc = bt_ref[0]
    g_iota = lax.broadcasted_iota(jnp.int32, (G, 1), 0)
    oht = (g_iota == br).astype(jnp.float32)

    @pl.when(i == 0)
    def _():
        sum_s[...] = jnp.zeros_like(sum_s)
        cnt_s[...] = jnp.zeros_like(cnt_s)
        max_s[...] = jnp.full_like(max_s, -3.4e38)

    sum_s[...] += jnp.dot(oht, xf, preferred_element_type=jnp.float32)
    cnt_s[...] += jnp.sum(oht, axis=1, keepdims=True)
    for g in range(G):
        xm = jnp.where(bc == g, xf, -3.4e38)
        max_s[g:g + 1, :] = jnp.maximum(
            max_s[g:g + 1, :], jnp.max(xm, axis=0, keepdims=True))

    @pl.when(i == _NGRID - 1)
    def _():
        cnt = jnp.maximum(cnt_s[...], 1.0)
        mean_p = sum_s[...] / cnt
        h = jnp.concatenate([mean_p, max_s[...], sum_s[...]], axis=1)
        h = jnp.maximum(
            jnp.dot(h, w1_ref[...], preferred_element_type=jnp.float32)
            + b1_ref[...], 0.0)
        h = jnp.maximum(
            jnp.dot(h, w2_ref[...], preferred_element_type=jnp.float32)
            + b2_ref[...], 0.0)
        out_ref[...] = (jnp.dot(h, w3_ref[...], preferred_element_type=jnp.float32)
                        + b3_ref[...])


def _final(x, agg, ssum, ssq, gam, bet, boh, bt, w1, b1, w2, b2, w3, b3):
    return pl.pallas_call(
        _final_body,
        grid=(_NGRID,),
        in_specs=[
            pl.BlockSpec((_NROW, F), lambda i: (i, 0)),
            pl.BlockSpec((_NROW, F), lambda i: (i, 0)),
            pl.BlockSpec((8, F), lambda i: (0, 0)),
            pl.BlockSpec((8, F), lambda i: (0, 0)),
            pl.BlockSpec((1, F), lambda i: (0, 0)),
            pl.BlockSpec((1, F), lambda i: (0, 0)),
            pl.BlockSpec((1, 1, _NROW), lambda i: (i, 0, 0)),
            pl.BlockSpec((1, _NROW, 1), lambda i: (i, 0, 0)),
            pl.BlockSpec((3 * F, 32), lambda i: (0, 0)),
            pl.BlockSpec((1, 32), lambda i: (0, 0)),
            pl.BlockSpec((32, 16), lambda i: (0, 0)),
            pl.BlockSpec((1, 16), lambda i: (0, 0)),
            pl.BlockSpec((16, 1), lambda i: (0, 0)),
            pl.BlockSpec((1, 1), lambda i: (0, 0)),
        ],
        out_specs=pl.BlockSpec((G, 1), lambda i: (0, 0)),
        out_shape=jax.ShapeDtypeStruct((G, 1), jnp.float32),
        scratch_shapes=[
            pltpu.VMEM((G, F), jnp.float32),
            pltpu.VMEM((G, F), jnp.float32),
            pltpu.VMEM((G, 1), jnp.float32),
        ],
    )(x, agg, ssum, ssq, gam, bet, boh, bt, w1, b1, w2, b2, w3, b3)


_LN1P = (0.9999811345733445, -0.49947171890986775, 0.32824174021996677,
         -0.22589647209838543, 0.13467197769209693, -0.05514315080324306,
         0.010763882328529595)


def _gate_rows(rows_d, rows_s, ef_v, msg_v, r):
    for v in range(F // 16):
        o = v * 16
        fa = rows_d[r, pl.ds(o, 16)] + rows_s[r, pl.ds(o, 16)] \
            + ef_v[r, pl.ds(o, 16)]
        sb = rows_d[r, pl.ds(F + o, 16)] + rows_s[r, pl.ds(F + o, 16)] \
            + ef_v[r, pl.ds(F + o, 16)]
        sg = 1.0 / (1.0 + jnp.exp(-fa))
        t = jnp.exp(-jnp.abs(sb))
        pl1 = _LN1P[6]
        for cc in _LN1P[5::-1]:
            pl1 = pl1 * t + cc
        sp = jnp.maximum(sb, 0.0) + t * pl1
        msg_v[r, pl.ds(o, 16)] = sg * sp


@functools.lru_cache(maxsize=None)
def _make_sc_agg_kernel():
    mesh = plsc.VectorSubcoreMesh(core_axis_name="c", subcore_axis_name="s")
    f32, i32 = jnp.float32, jnp.int32
    scratch = [
        pltpu.VMEM((CH,), i32), pltpu.VMEM((CH,), i32),
        pltpu.VMEM((CH,), i32), pltpu.VMEM((CH,), i32),
        pltpu.VMEM((CH,), i32), pltpu.VMEM((CH,), i32),
        pltpu.VMEM((CH, 2 * F), f32), pltpu.VMEM((CH, 2 * F), f32),
        pltpu.VMEM((CH, 2 * F), f32), pltpu.VMEM((CH, 2 * F), f32),
        pltpu.VMEM((CH, 2 * F), f32), pltpu.VMEM((CH, 2 * F), f32),
        pltpu.VMEM((CH, F), f32), pltpu.VMEM((CH, F), f32),
        pltpu.VMEM_SHARED((NPAD, F), f32),
    ] + [pltpu.SemaphoreType.DMA] * 12

    def body(dt_hbm, st_hbm, ef_hbm, d2_hbm, s2_hbm, z128_hbm, aggp_hbm,
             dib0, dib1, sib0, sib1, dsb0, dsb1,
             rd0, rd1, rs0, rs1, efv0, efv1, m0, m1, agg_sh,
             sid0, sid1, sis0, sis1, sd0, sd1, ss0, ss1, se0, se1, sc0, sc1):
        c = lax.axis_index("c")
        s = lax.axis_index("s")
        wid = s * 2 + c
        dib = (dib0, dib1)
        sib = (sib0, sib1)
        dsb = (dsb0, dsb1)
        rd = (rd0, rd1)
        rs = (rs0, rs1)
        efv = (efv0, efv1)
        m = (m0, m1)
        sid = (sid0, sid1)
        sis = (sis0, sis1)
        sd = (sd0, sd1)
        ss = (ss0, ss1)
        se = (se0, se1)
        sc = (sc0, sc1)
        cbase = wid * NCHUNK

        def issue_idx(ci, k):
            pltpu.async_copy(d2_hbm.at[cbase + ci], dib[k], sid[k])
            pltpu.async_copy(s2_hbm.at[cbase + ci], sib[k], sis[k])

        def wait_idx(k):
            pltpu.make_async_copy(d2_hbm.at[0], dib[k], sid[k]).wait()
            pltpu.make_async_copy(s2_hbm.at[0], sib[k], sis[k]).wait()

        def issue_gathers(ci, k):
            pltpu.async_copy(dt_hbm.at[dib[k]], rd[k], sd[k])
            pltpu.async_copy(st_hbm.at[sib[k]], rs[k], ss[k])
            pltpu.async_copy(ef_hbm.at[pl.ds((cbase + ci) * CH, CH)],
                             efv[k], se[k])

        def wait_gathers(k):
            pltpu.make_async_copy(dt_hbm.at[dib[k]], rd[k], sd[k]).wait()
            pltpu.make_async_copy(st_hbm.at[sib[k]], rs[k], ss[k]).wait()
            pltpu.make_async_copy(ef_hbm.at[pl.ds(0, CH)], efv[k], se[k]).wait()

        def issue_scatter(k):
            pltpu.async_copy(m[k], agg_sh.at[dsb[k]], sc[k], add=True)

        def wait_scatter(k):
            pltpu.make_async_copy(m[k], agg_sh.at[dsb[k]], sc[k]).wait()

        def compute(k):
            def row(r, carry):
                _gate_rows(rd[k], rs[k], efv[k], m[k], r)
                return carry

            lax.fori_loop(0, CH, row, 0)

        def step(ci, k, o, first, last):
            if not first:
                wait_scatter(o)
            if not last:
                wait_idx(o)
                issue_gathers(ci + 1, o)
            wait_gathers(k)
            dsb[k][...] = dib[k][...]
            if not last:
                @pl.when(ci + 2 < NCHUNK)
                def _():
                    issue_idx(ci + 2, k)
            compute(k)
            issue_scatter(k)

        pltpu.sync_copy(z128_hbm, agg_sh.at[pl.ds(s * ROWS_PER_SUB, ROWS_PER_SUB)])
        plsc.subcore_barrier()

        issue_idx(0, 0)
        wait_idx(0)
        issue_gathers(0, 0)
        issue_idx(1, 1)

        def pair(i, carry):
            c2 = i * 2

            @pl.when(i == 0)
            def _():
                step(c2, 0, 1, True, False)

            @pl.when(i > 0)
            def _():
                step(c2, 0, 1, False, False)

            step(c2 + 1, 1, 0, False, False)
            return carry

        lax.fori_loop(0, (NCHUNK - 1) // 2, pair, 0)
        step(NCHUNK - 1, 0, 1, False, True)
        wait_scatter(0)

        plsc.subcore_barrier()
        sl = pl.ds(s * ROWS_PER_SUB, ROWS_PER_SUB)
        pltpu.sync_copy(agg_sh.at[sl], aggp_hbm.at[c, sl])

    return pl.kernel(body, out_type=jax.ShapeDtypeStruct((2, NPAD, F), jnp.float32),
                     mesh=mesh, scratch_types=tuple(scratch))


@functools.lru_cache(maxsize=None)
def _make_sc_deg_kernel():
    mesh = plsc.VectorSubcoreMesh(core_axis_name="c", subcore_axis_name="s")
    scratch = [
        pltpu.VMEM((CHD,), jnp.int32),
        pltpu.VMEM((CHD, F), jnp.float32),
        pltpu.VMEM_SHARED((NPAD, F), jnp.float32),
    ]

    def body(di_hbm, z128_hbm, ones_hbm, degp_hbm, di_v, ones_v, deg_sh):
        c = lax.axis_index("c")
        s = lax.axis_index("s")
        wid = s * 2 + c
        pltpu.sync_copy(z128_hbm, deg_sh.at[pl.ds(s * ROWS_PER_SUB, ROWS_PER_SUB)])
        pltpu.sync_copy(ones_hbm, ones_v)
        plsc.subcore_barrier()

        def chunk(i, carry):
            base = wid * EPW + i * CHD
            pltpu.sync_copy(di_hbm.at[pl.ds(base, CHD)], di_v)
            pltpu.sync_copy(ones_v, deg_sh.at[di_v], add=True)
            return carry

        lax.fori_loop(0, NCHUNKD, chunk, 0)
        plsc.subcore_barrier()
        sl = pl.ds(s * ROWS_PER_SUB, ROWS_PER_SUB)
        pltpu.sync_copy(deg_sh.at[sl], degp_hbm.at[c, sl])

    return pl.kernel(body, out_type=jax.ShapeDtypeStruct((2, NPAD, F), jnp.float32),
                     mesh=mesh, scratch_types=tuple(scratch))


def _sc_deg(dsti, z128, ones):
    return _make_sc_deg_kernel()(dsti, z128, ones)


def _sc_agg(dt, st, ef, d2, s2, z128):
    return _make_sc_agg_kernel()(dt, st, ef, d2, s2, z128)


def kernel(x, edge_index, edge_attr, batch, Wf, bf, Ws, bs,
           bn_gamma, bn_beta, W1, b1, W2, b2, W3, b3):
    f32 = jnp.float32
    src = edge_index[0]
    dst = edge_index[1]

    centers = jnp.concatenate(
        [jnp.linspace(0.0, 5.0, 100, dtype=f32),
         jnp.full((28,), 1e9, dtype=f32)]).reshape(128, 1)
    wcat = jnp.pad(jnp.concatenate([Wf[:, 2 * F:, :], Ws[:, 2 * F:, :]], axis=2),
                   ((0, 0), (0, 28), (0, 0)))
    bcat = jnp.concatenate([bf, bs], axis=1).reshape(NB, 1, 2 * F)
    wd = jnp.concatenate([Wf[:, :F, :], Ws[:, :F, :]], axis=2)
    wsrc = jnp.concatenate([Wf[:, F:2 * F, :], Ws[:, F:2 * F, :]], axis=2)

    ea3 = edge_attr.reshape(_EA_GRID, 1, _EA_CHUNK)
    efs = _edge_feats(ea3, centers, wcat, bcat)

    z128 = jnp.zeros((ROWS_PER_SUB, F), f32)
    ones128 = jnp.ones((CHD, F), f32)

    boh = batch.reshape(_NGRID, 1, _NROW)
    bt = batch.reshape(_NGRID, _NROW, 1)

    d2 = dst.reshape(E // CH, CH)
    s2 = src.reshape(E // CH, CH)
    degp = _sc_deg(dst, z128, ones128)
    xcur = x
    aggp = None
    for b in range(NB):
        if b == 0:
            dt, st = _tables0(xcur, wd[0], wsrc[0])
        else:
            agg, ssum, ssq = _agg_stats(aggp, degp)
            xcur, dt, st = _tables_bn(
                xcur, agg, ssum, ssq,
                bn_gamma[b - 1].reshape(1, F), bn_beta[b - 1].reshape(1, F),
                wd[b], wsrc[b])
        aggp = _sc_agg(dt, st, efs[b], d2, s2, z128)

    agg, ssum, ssq = _agg_stats(aggp, degp)
    out = _final(xcur, agg, ssum, ssq,
                 bn_gamma[2].reshape(1, F), bn_beta[2].reshape(1, F),
                 boh, bt, W1, b1.reshape(1, 32), W2, b2.reshape(1, 16),
                 W3, b3.reshape(1, 1))
    return out


# speedup vs baseline: 1.3037x; 1.0243x over previous
import functools

import jax
import jax.numpy as jnp
from jax import lax
from jax.experimental import pallas as pl
from jax.experimental.pallas import tpu as pltpu
from jax.experimental.pallas import tpu_sc as plsc

N = 10000
E = 320000
F = 128
NB = 3
G = 64

NPAD = 10240
ROWS_PER_SUB = NPAD // 16
NW = 32
EPW = E // NW
CH = 16
NCHUNK = EPW // CH
CHD = 80
NCHUNKD = EPW // CHD

_EA_CHUNK = 512
_EA_GRID = E // _EA_CHUNK
_NROW = 1000
_NGRID = N // _NROW


def _edge_feat_body(ea_ref, cen_ref, wcat_ref, bcat_ref, o0_ref, o1_ref, o2_ref):
    d = ea_ref[0]
    t = (cen_ref[...] - d) * (99.0 / 5.0)
    e = jnp.exp(-(t * t))
    outs = (o0_ref, o1_ref, o2_ref)
    for b in range(NB):
        outs[b][...] = lax.dot_general(
            e, wcat_ref[b], (((0,), (0,)), ((), ())),
            preferred_element_type=jnp.float32) + bcat_ref[b]


def _edge_feats(ea3, centers, wcat, bcat):
    out = jax.ShapeDtypeStruct((E, 2 * F), jnp.float32)
    return pl.pallas_call(
        _edge_feat_body,
        grid=(_EA_GRID,),
        in_specs=[
            pl.BlockSpec((1, 1, _EA_CHUNK), lambda i: (i, 0, 0)),
            pl.BlockSpec((128, 1), lambda i: (0, 0)),
            pl.BlockSpec((NB, 128, 2 * F), lambda i: (0, 0, 0)),
            pl.BlockSpec((NB, 1, 2 * F), lambda i: (0, 0, 0)),
        ],
        out_specs=[pl.BlockSpec((_EA_CHUNK, 2 * F), lambda i: (i, 0))] * NB,
        out_shape=[out, out, out],
    )(ea3, centers, wcat, bcat)


def _tables0_body(x_ref, wd_ref, wsrc_ref, dt_ref, st_ref):
    xb = x_ref[...]
    dt_ref[...] = jnp.dot(xb, wd_ref[...], preferred_element_type=jnp.float32)
    st_ref[...] = jnp.dot(xb, wsrc_ref[...], preferred_element_type=jnp.float32)


def _tables0(x, wd, wsrc):
    out = jax.ShapeDtypeStruct((N, 2 * F), jnp.float32)
    return pl.pallas_call(
        _tables0_body,
        grid=(_NGRID,),
        in_specs=[
            pl.BlockSpec((_NROW, F), lambda i: (i, 0)),
            pl.BlockSpec((F, 2 * F), lambda i: (0, 0)),
            pl.BlockSpec((F, 2 * F), lambda i: (0, 0)),
        ],
        out_specs=[pl.BlockSpec((_NROW, 2 * F), lambda i: (i, 0))] * 2,
        out_shape=[out, out],
    )(x, wd, wsrc)


def _tables_bn_body(x_ref, agg_ref, ssum_ref, ssq_ref, gam_ref, bet_ref,
                    wd_ref, wsrc_ref, xn_ref, dt_ref, st_ref):
    mu = ssum_ref[0:1, :] * (1.0 / N)
    ex2 = ssq_ref[0:1, :] * (1.0 / N)
    scv = gam_ref[...] * lax.rsqrt(ex2 - mu * mu + 1e-5)
    xn = x_ref[...] + (agg_ref[...] - mu) * scv + bet_ref[...]
    xn_ref[...] = xn
    dt_ref[...] = jnp.dot(xn, wd_ref[...], preferred_element_type=jnp.float32)
    st_ref[...] = jnp.dot(xn, wsrc_ref[...], preferred_element_type=jnp.float32)


def _tables_bn(x, agg, ssum, ssq, gam, bet, wd, wsrc):
    out2 = jax.ShapeDtypeStruct((N, 2 * F), jnp.float32)
    outx = jax.ShapeDtypeStruct((N, F), jnp.float32)
    return pl.pallas_call(
        _tables_bn_body,
        grid=(_NGRID,),
        in_specs=[
            pl.BlockSpec((_NROW, F), lambda i: (i, 0)),
            pl.BlockSpec((_NROW, F), lambda i: (i, 0)),
            pl.BlockSpec((8, F), lambda i: (0, 0)),
            pl.BlockSpec((8, F), lambda i: (0, 0)),
            pl.BlockSpec((1, F), lambda i: (0, 0)),
            pl.BlockSpec((1, F), lambda i: (0, 0)),
            pl.BlockSpec((F, 2 * F), lambda i: (0, 0)),
            pl.BlockSpec((F, 2 * F), lambda i: (0, 0)),
        ],
        out_specs=[
            pl.BlockSpec((_NROW, F), lambda i: (i, 0)),
            pl.BlockSpec((_NROW, 2 * F), lambda i: (i, 0)),
            pl.BlockSpec((_NROW, 2 * F), lambda i: (i, 0)),
        ],
        out_shape=[outx, out2, out2],
    )(x, agg, ssum, ssq, gam, bet, wd, wsrc)


def _agg_stats_body(p_ref, dp_ref, agg_ref, ssum_ref, ssq_ref):
    i = pl.program_id(0)
    deg = jnp.maximum(dp_ref[0, :, 0:1] + dp_ref[1, :, 0:1], 1.0)
    agg = (p_ref[0] + p_ref[1]) / deg
    agg_ref[...] = agg
    s = jnp.sum(agg, axis=0, keepdims=True)
    s2 = jnp.sum(agg * agg, axis=0, keepdims=True)

    @pl.when(i == 0)
    def _():
        ssum_ref[...] = jnp.zeros_like(ssum_ref)
        ssq_ref[...] = jnp.zeros_like(ssq_ref)

    ssum_ref[...] += jnp.broadcast_to(s, (8, F))
    ssq_ref[...] += jnp.broadcast_to(s2, (8, F))


def _agg_stats(aggp, degp):
    return pl.pallas_call(
        _agg_stats_body,
        grid=(_NGRID,),
        in_specs=[
            pl.BlockSpec((2, _NROW, F), lambda i: (0, i, 0)),
            pl.BlockSpec((2, _NROW, F), lambda i: (0, i, 0)),
        ],
        out_specs=[
            pl.BlockSpec((_NROW, F), lambda i: (i, 0)),
            pl.BlockSpec((8, F), lambda i: (0, 0)),
            pl.BlockSpec((8, F), lambda i: (0, 0)),
        ],
        out_shape=[
            jax.ShapeDtypeStruct((N, F), jnp.float32),
            jax.ShapeDtypeStruct((8, F), jnp.float32),
            jax.ShapeDtypeStruct((8, F), jnp.float32),
        ],
    )(aggp, degp)


def _final_body(x_ref, agg_ref, ssum_ref, ssq_ref, gam_ref, bet_ref,
                boh_ref, bt_ref, w1_ref, b1_ref, w2_ref, b2_ref,
                w3_ref, b3_ref, out_ref, sum_s, max_s, cnt_s):
    i = pl.program_id(0)
    mu = ssum_ref[0:1, :] * (1.0 / N)
    ex2 = ssq_ref[0:1, :] * (1.0 / N)
    scv = gam_ref[...] * lax.rsqrt(ex2 - mu * mu + 1e-5)
    xf = x_ref[...] + (agg_ref[...] - mu) * scv + bet_ref[...]

    br = boh_ref[0]
    bc = bt_ref[0]
    g_iota = lax.broadcasted_iota(jnp.int32, (G, 1), 0)
    oht = (g_iota == br).astype(jnp.float32)

    @pl.when(i == 0)
    def _():
        sum_s[...] = jnp.zeros_like(sum_s)
        cnt_s[...] = jnp.zeros_like(cnt_s)
        max_s[...] = jnp.full_like(max_s, -3.4e38)

    sum_s[...] += jnp.dot(oht, xf, preferred_element_type=jnp.float32)
    cnt_s[...] += jnp.sum(oht, axis=1, keepdims=True)
    for g in range(G):
        xm = jnp.where(bc == g, xf, -3.4e38)
        max_s[g:g + 1, :] = jnp.maximum(
            max_s[g:g + 1, :], jnp.max(xm, axis=0, keepdims=True))

    @pl.when(i == _NGRID - 1)
    def _():
        cnt = jnp.maximum(cnt_s[...], 1.0)
        mean_p = sum_s[...] / cnt
        h = jnp.concatenate([mean_p, max_s[...], sum_s[...]], axis=1)
        h = jnp.maximum(
            jnp.dot(h, w1_ref[...], preferred_element_type=jnp.float32)
            + b1_ref[...], 0.0)
        h = jnp.maximum(
            jnp.dot(h, w2_ref[...], preferred_element_type=jnp.float32)
            + b2_ref[...], 0.0)
        out_ref[...] = (jnp.dot(h, w3_ref[...], preferred_element_type=jnp.float32)
                        + b3_ref[...])


def _final(x, agg, ssum, ssq, gam, bet, boh, bt, w1, b1, w2, b2, w3, b3):
    return pl.pallas_call(
        _final_body,
        grid=(_NGRID,),
        in_specs=[
            pl.BlockSpec((_NROW, F), lambda i: (i, 0)),
            pl.BlockSpec((_NROW, F), lambda i: (i, 0)),
            pl.BlockSpec((8, F), lambda i: (0, 0)),
            pl.BlockSpec((8, F), lambda i: (0, 0)),
            pl.BlockSpec((1, F), lambda i: (0, 0)),
            pl.BlockSpec((1, F), lambda i: (0, 0)),
            pl.BlockSpec((1, 1, _NROW), lambda i: (i, 0, 0)),
            pl.BlockSpec((1, _NROW, 1), lambda i: (i, 0, 0)),
            pl.BlockSpec((3 * F, 32), lambda i: (0, 0)),
            pl.BlockSpec((1, 32), lambda i: (0, 0)),
            pl.BlockSpec((32, 16), lambda i: (0, 0)),
            pl.BlockSpec((1, 16), lambda i: (0, 0)),
            pl.BlockSpec((16, 1), lambda i: (0, 0)),
            pl.BlockSpec((1, 1), lambda i: (0, 0)),
        ],
        out_specs=pl.BlockSpec((G, 1), lambda i: (0, 0)),
        out_shape=jax.ShapeDtypeStruct((G, 1), jnp.float32),
        scratch_shapes=[
            pltpu.VMEM((G, F), jnp.float32),
            pltpu.VMEM((G, F), jnp.float32),
            pltpu.VMEM((G, 1), jnp.float32),
        ],
    )(x, agg, ssum, ssq, gam, bet, boh, bt, w1, b1, w2, b2, w3, b3)


_LN1P = (0.9999811345733445, -0.49947171890986775, 0.32824174021996677,
         -0.22589647209838543, 0.13467197769209693, -0.05514315080324306,
         0.010763882328529595)


def _gate_rows(rows_d, rows_s, ef_v, msg_v, r):
    for v in range(F // 16):
        o = v * 16
        fa = rows_d[r, pl.ds(o, 16)] + rows_s[r, pl.ds(o, 16)] \
            + ef_v[r, pl.ds(o, 16)]
        sb = rows_d[r, pl.ds(F + o, 16)] + rows_s[r, pl.ds(F + o, 16)] \
            + ef_v[r, pl.ds(F + o, 16)]
        sg = 1.0 / (1.0 + jnp.exp(-fa))
        t = jnp.exp(-jnp.abs(sb))
        pl1 = _LN1P[6]
        for cc in _LN1P[5::-1]:
            pl1 = pl1 * t + cc
        sp = jnp.maximum(sb, 0.0) + t * pl1
        msg_v[r, pl.ds(o, 16)] = sg * sp


@functools.lru_cache(maxsize=None)
def _make_sc_agg_kernel():
    mesh = plsc.VectorSubcoreMesh(core_axis_name="c", subcore_axis_name="s")
    f32, i32 = jnp.float32, jnp.int32
    scratch = [
        pltpu.VMEM((CH,), i32), pltpu.VMEM((CH,), i32),
        pltpu.VMEM((CH,), i32), pltpu.VMEM((CH,), i32),
        pltpu.VMEM((CH,), i32), pltpu.VMEM((CH,), i32),
        pltpu.VMEM((CH, 2 * F), f32), pltpu.VMEM((CH, 2 * F), f32),
        pltpu.VMEM((CH, 2 * F), f32), pltpu.VMEM((CH, 2 * F), f32),
        pltpu.VMEM((CH, 2 * F), f32), pltpu.VMEM((CH, 2 * F), f32),
        pltpu.VMEM((CH, F), f32), pltpu.VMEM((CH, F), f32),
        pltpu.VMEM_SHARED((NPAD, F), f32),
    ] + [pltpu.SemaphoreType.DMA] * 12

    def body(dt_hbm, st_hbm, ef_hbm, d2_hbm, s2_hbm, z128_hbm, aggp_hbm,
             dib0, dib1, sib0, sib1, dsb0, dsb1,
             rd0, rd1, rs0, rs1, efv0, efv1, m0, m1, agg_sh,
             sid0, sid1, sis0, sis1, sd0, sd1, ss0, ss1, se0, se1, sc0, sc1):
        c = lax.axis_index("c")
        s = lax.axis_index("s")
        wid = s * 2 + c
        dib = (dib0, dib1)
        sib = (sib0, sib1)
        dsb = (dsb0, dsb1)
        rd = (rd0, rd1)
        rs = (rs0, rs1)
        efv = (efv0, efv1)
        m = (m0, m1)
        sid = (sid0, sid1)
        sis = (sis0, sis1)
        sd = (sd0, sd1)
        ss = (ss0, ss1)
        se = (se0, se1)
        sc = (sc0, sc1)
        cbase = wid * NCHUNK

        def issue_idx(ci, k):
            pltpu.async_copy(d2_hbm.at[cbase + ci], dib[k], sid[k])
            pltpu.async_copy(s2_hbm.at[cbase + ci], sib[k], sis[k])

        def wait_idx(k):
            pltpu.make_async_copy(d2_hbm.at[0], dib[k], sid[k]).wait()
            pltpu.make_async_copy(s2_hbm.at[0], sib[k], sis[k]).wait()

        def issue_gathers(ci, k):
            pltpu.async_copy(dt_hbm.at[dib[k]], rd[k], sd[k])
            pltpu.async_copy(st_hbm.at[sib[k]], rs[k], ss[k])
            pltpu.async_copy(ef_hbm.at[pl.ds((cbase + ci) * CH, CH)],
                             efv[k], se[k])

        def wait_gathers(k):
            pltpu.make_async_copy(dt_hbm.at[dib[k]], rd[k], sd[k]).wait()
            pltpu.make_async_copy(st_hbm.at[sib[k]], rs[k], ss[k]).wait()
            pltpu.make_async_copy(ef_hbm.at[pl.ds(0, CH)], efv[k], se[k]).wait()

        def issue_scatter(k):
            pltpu.async_copy(m[k], agg_sh.at[dsb[k]], sc[k], add=True)

        def wait_scatter(k):
            pltpu.make_async_copy(m[k], agg_sh.at[dsb[k]], sc[k]).wait()

        def compute(k):
            @plsc.parallel_loop(0, CH, 1, unroll=4)
            def _(r):
                _gate_rows(rd[k], rs[k], efv[k], m[k], r)

        def step(ci, k, o, first, last):
            if not first:
                wait_scatter(o)
            if not last:
                wait_idx(o)
                issue_gathers(ci + 1, o)
            wait_gathers(k)
            dsb[k][...] = dib[k][...]
            if not last:
                @pl.when(ci + 2 < NCHUNK)
                def _():
                    issue_idx(ci + 2, k)
            compute(k)
            issue_scatter(k)

        pltpu.sync_copy(z128_hbm, agg_sh.at[pl.ds(s * ROWS_PER_SUB, ROWS_PER_SUB)])
        plsc.subcore_barrier()

        issue_idx(0, 0)
        wait_idx(0)
        issue_gathers(0, 0)
        issue_idx(1, 1)

        def pair(i, carry):
            c2 = i * 2

            @pl.when(i == 0)
            def _():
                step(c2, 0, 1, True, False)

            @pl.when(i > 0)
            def _():
                step(c2, 0, 1, False, False)

            step(c2 + 1, 1, 0, False, False)
            return carry

        lax.fori_loop(0, (NCHUNK - 1) // 2, pair, 0)
        step(NCHUNK - 1, 0, 1, False, True)
        wait_scatter(0)

        plsc.subcore_barrier()
        sl = pl.ds(s * ROWS_PER_SUB, ROWS_PER_SUB)
        pltpu.sync_copy(agg_sh.at[sl], aggp_hbm.at[c, sl])

    return pl.kernel(body, out_type=jax.ShapeDtypeStruct((2, NPAD, F), jnp.float32),
                     mesh=mesh, scratch_types=tuple(scratch))


@functools.lru_cache(maxsize=None)
def _make_sc_deg_kernel():
    mesh = plsc.VectorSubcoreMesh(core_axis_name="c", subcore_axis_name="s")
    scratch = [
        pltpu.VMEM((CHD,), jnp.int32),
        pltpu.VMEM((CHD, F), jnp.float32),
        pltpu.VMEM_SHARED((NPAD, F), jnp.float32),
    ]

    def body(di_hbm, z128_hbm, ones_hbm, degp_hbm, di_v, ones_v, deg_sh):
        c = lax.axis_index("c")
        s = lax.axis_index("s")
        wid = s * 2 + c
        pltpu.sync_copy(z128_hbm, deg_sh.at[pl.ds(s * ROWS_PER_SUB, ROWS_PER_SUB)])
        pltpu.sync_copy(ones_hbm, ones_v)
        plsc.subcore_barrier()

        def chunk(i, carry):
            base = wid * EPW + i * CHD
            pltpu.sync_copy(di_hbm.at[pl.ds(base, CHD)], di_v)
            pltpu.sync_copy(ones_v, deg_sh.at[di_v], add=True)
            return carry

        lax.fori_loop(0, NCHUNKD, chunk, 0)
        plsc.subcore_barrier()
        sl = pl.ds(s * ROWS_PER_SUB, ROWS_PER_SUB)
        pltpu.sync_copy(deg_sh.at[sl], degp_hbm.at[c, sl])

    return pl.kernel(body, out_type=jax.ShapeDtypeStruct((2, NPAD, F), jnp.float32),
                     mesh=mesh, scratch_types=tuple(scratch))


def _sc_deg(dsti, z128, ones):
    return _make_sc_deg_kernel()(dsti, z128, ones)


def _sc_agg(dt, st, ef, d2, s2, z128):
    return _make_sc_agg_kernel()(dt, st, ef, d2, s2, z128)


def kernel(x, edge_index, edge_attr, batch, Wf, bf, Ws, bs,
           bn_gamma, bn_beta, W1, b1, W2, b2, W3, b3):
    f32 = jnp.float32
    src = edge_index[0]
    dst = edge_index[1]

    centers = jnp.concatenate(
        [jnp.linspace(0.0, 5.0, 100, dtype=f32),
         jnp.full((28,), 1e9, dtype=f32)]).reshape(128, 1)
    wcat = jnp.pad(jnp.concatenate([Wf[:, 2 * F:, :], Ws[:, 2 * F:, :]], axis=2),
                   ((0, 0), (0, 28), (0, 0)))
    bcat = jnp.concatenate([bf, bs], axis=1).reshape(NB, 1, 2 * F)
    wd = jnp.concatenate([Wf[:, :F, :], Ws[:, :F, :]], axis=2)
    wsrc = jnp.concatenate([Wf[:, F:2 * F, :], Ws[:, F:2 * F, :]], axis=2)

    ea3 = edge_attr.reshape(_EA_GRID, 1, _EA_CHUNK)
    efs = _edge_feats(ea3, centers, wcat, bcat)

    z128 = jnp.zeros((ROWS_PER_SUB, F), f32)
    ones128 = jnp.ones((CHD, F), f32)

    boh = batch.reshape(_NGRID, 1, _NROW)
    bt = batch.reshape(_NGRID, _NROW, 1)

    d2 = dst.reshape(E // CH, CH)
    s2 = src.reshape(E // CH, CH)
    degp = _sc_deg(dst, z128, ones128)
    xcur = x
    aggp = None
    for b in range(NB):
        if b == 0:
            dt, st = _tables0(xcur, wd[0], wsrc[0])
        else:
            agg, ssum, ssq = _agg_stats(aggp, degp)
            xcur, dt, st = _tables_bn(
                xcur, agg, ssum, ssq,
                bn_gamma[b - 1].reshape(1, F), bn_beta[b - 1].reshape(1, F),
                wd[b], wsrc[b])
        aggp = _sc_agg(dt, st, efs[b], d2, s2, z128)

    agg, ssum, ssq = _agg_stats(aggp, degp)
    out = _final(xcur, agg, ssum, ssq,
                 bn_gamma[2].reshape(1, F), bn_beta[2].reshape(1, F),
                 boh, bt, W1, b1.reshape(1, 32), W2, b2.reshape(1, 16),
                 W3, b3.reshape(1, 1))
    return out
